# baseline (speedup 1.0000x reference)
import functools

import jax
import jax.numpy as jnp
from jax import lax
from jax.experimental import pallas as pl
from jax.experimental.pallas import tpu as pltpu
from jax.experimental.pallas import tpu_sc as plsc

CAP = 8192
ROW = 8 * 128
NEW = 32
NC, NS = 2, 16
NW = NC * NS
RPW = CAP // NW

_mesh = plsc.VectorSubcoreMesh(
    core_axis_name="c", subcore_axis_name="s", num_cores=NC, num_subcores=NS
)


@functools.partial(
    pl.kernel,
    out_type=(
        jax.ShapeDtypeStruct((CAP, ROW), jnp.float32),
        jax.ShapeDtypeStruct((CAP, ROW), jnp.float32),
    ),
    mesh=_mesh,
    scratch_types=[
        pltpu.VMEM((16,), jnp.int32),
        pltpu.SemaphoreType.DMA,
        pltpu.SemaphoreType.DMA,
    ],
)
def _extend(keys, values, cl_vec, new_keys, new_values,
            out_k, out_v, cl_vmem, sem_k, sem_v):
    wid = lax.axis_index("c") * NS + lax.axis_index("s")
    base = wid * RPW

    pltpu.sync_copy(cl_vec, cl_vmem)
    cl = cl_vmem[...][0]

    cp_k = pltpu.async_copy(keys.at[pl.ds(base, RPW)],
                            out_k.at[pl.ds(base, RPW)], sem_k)
    cp_v = pltpu.async_copy(values.at[pl.ds(base, RPW)],
                            out_v.at[pl.ds(base, RPW)], sem_v)
    cp_k.wait()
    cp_v.wait()

    def body(r, carry):
        dest = cl + r

        @pl.when(jnp.logical_and(dest >= base, dest < base + RPW))
        def _():
            pltpu.sync_copy(new_keys.at[pl.ds(r, 1)], out_k.at[pl.ds(dest, 1)])
            pltpu.sync_copy(new_values.at[pl.ds(r, 1)], out_v.at[pl.ds(dest, 1)])

        return carry

    lax.fori_loop(0, NEW, body, 0)


def kernel(keys, values, current_length, new_keys, new_values):
    k2 = keys.reshape(CAP, ROW)
    v2 = values.reshape(CAP, ROW)
    nk2 = new_keys.reshape(NEW, ROW)
    nv2 = new_values.reshape(NEW, ROW)
    cl_vec = jnp.full((16,), current_length, dtype=jnp.int32)
    out_k, out_v = _extend(k2, v2, cl_vec, nk2, nv2)
    return (out_k.reshape(keys.shape), out_v.reshape(values.shape),
            current_length + NEW)


# speedup vs baseline: 1.0414x; 1.0414x over previous
import functools

import jax
import jax.numpy as jnp
from jax import lax
from jax.experimental import pallas as pl
from jax.experimental.pallas import tpu as pltpu
from jax.experimental.pallas import tpu_sc as plsc

CAP = 8192
ROW = 8 * 128
NEW = 32
NC, NS = 2, 16
NW = NC * NS
RPW = CAP // NW
FLAT = CAP * ROW
EPW = FLAT // NW

_mesh = plsc.VectorSubcoreMesh(
    core_axis_name="c", subcore_axis_name="s", num_cores=NC, num_subcores=NS
)


@functools.partial(
    pl.kernel,
    out_type=(
        jax.ShapeDtypeStruct((FLAT,), jnp.float32),
        jax.ShapeDtypeStruct((FLAT,), jnp.float32),
    ),
    mesh=_mesh,
    scratch_types=[
        pltpu.VMEM((16,), jnp.int32),
        pltpu.SemaphoreType.DMA,
        pltpu.SemaphoreType.DMA,
    ],
)
def _extend(keys, values, cl_vec, new_keys, new_values,
            out_k, out_v, cl_vmem, sem_k, sem_v):
    wid = lax.axis_index("c") * NS + lax.axis_index("s")
    base = wid * EPW

    pltpu.sync_copy(cl_vec, cl_vmem)
    cl = cl_vmem[...][0]

    cp_k = pltpu.async_copy(keys.at[pl.ds(base, EPW)],
                            out_k.at[pl.ds(base, EPW)], sem_k)
    cp_v = pltpu.async_copy(values.at[pl.ds(base, EPW)],
                            out_v.at[pl.ds(base, EPW)], sem_v)
    cp_k.wait()
    cp_v.wait()

    def body(r, carry):
        dest = (cl + r) * ROW

        @pl.when(jnp.logical_and(dest >= base, dest < base + EPW))
        def _():
            pltpu.sync_copy(new_keys.at[pl.ds(r * ROW, ROW)],
                            out_k.at[pl.ds(dest, ROW)])
            pltpu.sync_copy(new_values.at[pl.ds(r * ROW, ROW)],
                            out_v.at[pl.ds(dest, ROW)])

        return carry

    lax.fori_loop(0, NEW, body, 0)


def kernel(keys, values, current_length, new_keys, new_values):
    k2 = keys.reshape(FLAT)
    v2 = values.reshape(FLAT)
    nk2 = new_keys.reshape(NEW * ROW)
    nv2 = new_values.reshape(NEW * ROW)
    cl_vec = jnp.full((16,), current_length, dtype=jnp.int32)
    out_k, out_v = _extend(k2, v2, cl_vec, nk2, nv2)
    return (out_k.reshape(keys.shape), out_v.reshape(values.shape),
            current_length + NEW)
